# baseline (speedup 1.0000x reference)
import functools

import jax
import jax.numpy as jnp
from jax import lax
from jax.experimental import pallas as pl
from jax.experimental.pallas import tpu as pltpu
import jax.experimental.pallas.tpu_sc as plsc

N_ENT = 50000
N_USR = 10000
N_TOT = 60000
D = 64
HD = 32
NREL = 32
E = 800000
BATCH = 1024

GROUP = 128
G_TOTAL = E // GROUP
G_PER_SC = G_TOTAL // 2
NTILE = 16
G_BASE = G_PER_SC // NTILE
G_REM = G_PER_SC % NTILE
ZROWS = 104
ZDEN = 520
ENT_RPT = 3128
TOT_RPT = 3752
N_PENT = ENT_RPT * NTILE
N_PTOT = TOT_RPT * NTILE
DEN_RPT = ENT_RPT
N_DEN = N_PENT

_mesh = lambda: plsc.VectorSubcoreMesh(core_axis_name="c", subcore_axis_name="s")


def _zero_z2d(z2d):
    def zb(i, carry):
        z2d[i, pl.ds(0, 16)] = jnp.zeros((16,), jnp.float32)
        z2d[i, pl.ds(16, 16)] = jnp.zeros((16,), jnp.float32)
        return carry
    lax.fori_loop(0, ZROWS, zb, 0)


def _zero_acc(z2d, acc_sh, row0, rpt):
    nz = rpt // ZROWS
    def zs(i, carry):
        pltpu.sync_copy(z2d, acc_sh.at[pl.ds(row0 + i * ZROWS, ZROWS)])
        return carry
    lax.fori_loop(0, nz, zs, 0)
    pltpu.sync_copy(z2d.at[pl.ds(0, 8)], acc_sh.at[pl.ds(row0 + rpt - 8, 8)])


def _group_span(c, s):
    lo_t = s * G_BASE + jnp.minimum(s, G_REM)
    cnt = G_BASE + jnp.where(s < G_REM, 1, 0)
    g0 = c * G_PER_SC + lo_t
    return g0, cnt


def _kg1_body(h_hbm, t_hbm, r_hbm, tf_hbm, hf_hbm, elo_hbm,
              w_hbm, num_out, den_out0, den_out1,
              h_idx, t_idx, r_idx, ti, hi2, av, bv, wv, rows, sx,
              num_sh, den_sh, sem_e, sem_g, sem_s, sem_w):
    c = lax.axis_index("c")
    s = lax.axis_index("s")
    row0 = s * ENT_RPT

    z16 = jnp.zeros((16,), jnp.float32)
    def zr(i, carry):
        rows[0][i, pl.ds(0, 16)] = z16
        rows[0][i, pl.ds(16, 16)] = z16
        return carry
    lax.fori_loop(0, GROUP, zr, 0)
    for k in range(GROUP // 16):
        wv[0][pl.ds(k * 16, 16)] = z16

    def za(i, carry):
        pltpu.sync_copy(rows[0], num_sh.at[pl.ds(row0 + i * GROUP, GROUP)])
        return carry
    lax.fori_loop(0, ENT_RPT // GROUP, za, 0)
    pltpu.sync_copy(rows[0], num_sh.at[pl.ds(row0 + ENT_RPT - GROUP, GROUP)])

    def zd(i, carry):
        pltpu.sync_copy(wv[0], den_sh.at[pl.ds(row0 + i * GROUP, GROUP)])
        return carry
    lax.fori_loop(0, DEN_RPT // GROUP, zd, 0)
    pltpu.sync_copy(wv[0], den_sh.at[pl.ds(row0 + DEN_RPT - GROUP, GROUP)])
    plsc.subcore_barrier()

    g0, cnt = _group_span(c, s)

    def fire_edge(g, b):
        base = (g0 + g) * GROUP
        pltpu.async_copy(h_hbm.at[pl.ds(base, GROUP)], h_idx[b], sem_e[b])
        pltpu.async_copy(t_hbm.at[pl.ds(base, GROUP)], t_idx[b], sem_e[b])
        pltpu.async_copy(r_hbm.at[pl.ds(base, GROUP)], r_idx[b], sem_e[b])

    def wait_edge(b):
        pltpu.make_async_copy(h_hbm.at[pl.ds(0, GROUP)], h_idx[b], sem_e[b]).wait()
        pltpu.make_async_copy(h_hbm.at[pl.ds(0, GROUP)], t_idx[b], sem_e[b]).wait()
        pltpu.make_async_copy(h_hbm.at[pl.ds(0, GROUP)], r_idx[b], sem_e[b]).wait()

    def wait_gath(b):
        pltpu.make_async_copy(tf_hbm.at[ti[b]], av[b], sem_g[b]).wait()
        pltpu.make_async_copy(hf_hbm.at[hi2[b]], bv[b], sem_g[b]).wait()
        pltpu.make_async_copy(elo_hbm.at[t_idx[b]], rows[b], sem_g[b]).wait()

    def wait_scat(b):
        pltpu.make_async_copy(rows[b], num_sh.at[sx[b]], sem_s[b]).wait()
        pltpu.make_async_copy(wv[b], den_sh.at[sx[b]], sem_s[b]).wait()
        pltpu.make_async_copy(wv[b], w_hbm.at[pl.ds(0, GROUP)], sem_w[b]).wait()

    fire_edge(0, 0)

    def grp2(i, carry):
        for par in range(2):
            g = i * 2 + par
            b = par
            o = 1 - par

            @pl.when(g < cnt)
            def _():
                wait_edge(b)
                for k in range(GROUP // 16):
                    sl = pl.ds(k * 16, 16)
                    rr = r_idx[b][sl]
                    ti[b][sl] = t_idx[b][sl] * NREL + rr
                    hi2[b][sl] = h_idx[b][sl] * NREL + rr

                @pl.when(g >= 2)
                def _():
                    wait_scat(b)
                pltpu.async_copy(tf_hbm.at[ti[b]], av[b], sem_g[b])
                pltpu.async_copy(hf_hbm.at[hi2[b]], bv[b], sem_g[b])
                pltpu.async_copy(elo_hbm.at[t_idx[b]], rows[b], sem_g[b])

            @pl.when((g >= 1) & (g <= cnt))
            def _():
                wait_gath(o)
                for k in range(GROUP // 16):
                    sl = pl.ds(k * 16, 16)
                    v = av[o][sl] + bv[o][sl]
                    v = jnp.maximum(v, v * 0.01)
                    wv[o][sl] = jnp.exp(v)
                    sx[o][sl] = h_idx[o][sl]
                for k in range(GROUP // 16):
                    w16 = wv[o][pl.ds(k * 16, 16)]
                    for m in range(16):
                        e = k * 16 + m
                        we = w16[m]
                        rows[o][e, pl.ds(0, 16)] = rows[o][e, pl.ds(0, 16)] * we
                        rows[o][e, pl.ds(16, 16)] = rows[o][e, pl.ds(16, 16)] * we
                base_prev = (g0 + g - 1) * GROUP
                pltpu.async_copy(rows[o], num_sh.at[sx[o]], sem_s[o], add=True)
                pltpu.async_copy(wv[o], den_sh.at[sx[o]], sem_s[o], add=True)
                pltpu.async_copy(wv[o], w_hbm.at[pl.ds(base_prev, GROUP)],
                                 sem_w[o])

            @pl.when(g + 1 < cnt)
            def _():
                fire_edge(g + 1, o)
        return carry
    lax.fori_loop(0, (G_BASE + 2 + 1) // 2, grp2, 0)

    wait_scat(0)
    wait_scat(1)
    plsc.subcore_barrier()
    pltpu.sync_copy(num_sh.at[pl.ds(row0, ENT_RPT)],
                    num_out.at[c, pl.ds(row0, ENT_RPT)])

    @pl.when(c == 0)
    def _():
        pltpu.sync_copy(den_sh.at[pl.ds(row0, DEN_RPT)],
                        den_out0.at[pl.ds(row0, DEN_RPT)])

    @pl.when(c == 1)
    def _():
        pltpu.sync_copy(den_sh.at[pl.ds(row0, DEN_RPT)],
                        den_out1.at[pl.ds(row0, DEN_RPT)])


def _kg1(*args):
    pair = lambda sh, dt: (pltpu.VMEM(sh, dt), pltpu.VMEM(sh, dt))
    sems = lambda: (pltpu.SemaphoreType.DMA, pltpu.SemaphoreType.DMA)
    return pl.kernel(
        _kg1_body,
        out_type=[
            jax.ShapeDtypeStruct((E,), jnp.float32),
            jax.ShapeDtypeStruct((2, N_PENT, HD), jnp.float32),
            jax.ShapeDtypeStruct((N_DEN,), jnp.float32),
            jax.ShapeDtypeStruct((N_DEN,), jnp.float32),
        ],
        mesh=_mesh(),
        compiler_params=pltpu.CompilerParams(use_tc_tiling_on_sc=False),
        scratch_types=[
            pair((GROUP,), jnp.int32),
            pair((GROUP,), jnp.int32),
            pair((GROUP,), jnp.int32),
            pair((GROUP,), jnp.int32),
            pair((GROUP,), jnp.int32),
            pair((GROUP,), jnp.float32),
            pair((GROUP,), jnp.float32),
            pair((GROUP,), jnp.float32),
            pair((GROUP, HD), jnp.float32),
            pair((GROUP,), jnp.int32),
            pltpu.VMEM_SHARED((N_PENT, HD), jnp.float32),
            pltpu.VMEM_SHARED((N_DEN,), jnp.float32),
            sems(),
            sems(),
            sems(),
            sems(),
        ],
    )(*args)


def _ws_body(rpt, col_hbm, row_hbm, val_hbm, tab_hbm, acc_out,
             c_idx, r_idx, vv, rows, sx, acc_sh, sem_e, sem_g, sem_s):
    c = lax.axis_index("c")
    s = lax.axis_index("s")
    row0 = s * rpt

    z16 = jnp.zeros((16,), jnp.float32)
    def zr(i, carry):
        rows[0][i, pl.ds(0, 16)] = z16
        rows[0][i, pl.ds(16, 16)] = z16
        return carry
    lax.fori_loop(0, GROUP, zr, 0)

    def za(i, carry):
        pltpu.sync_copy(rows[0], acc_sh.at[pl.ds(row0 + i * GROUP, GROUP)])
        return carry
    lax.fori_loop(0, rpt // GROUP, za, 0)
    pltpu.sync_copy(rows[0], acc_sh.at[pl.ds(row0 + rpt - GROUP, GROUP)])
    plsc.subcore_barrier()

    g0, cnt = _group_span(c, s)

    def fire_edge(g, b):
        base = (g0 + g) * GROUP
        pltpu.async_copy(col_hbm.at[pl.ds(base, GROUP)], c_idx[b], sem_e[b])
        pltpu.async_copy(row_hbm.at[pl.ds(base, GROUP)], r_idx[b], sem_e[b])
        pltpu.async_copy(val_hbm.at[pl.ds(base, GROUP)], vv[b], sem_e[b])

    def wait_edge(b):
        pltpu.make_async_copy(col_hbm.at[pl.ds(0, GROUP)], c_idx[b], sem_e[b]).wait()
        pltpu.make_async_copy(col_hbm.at[pl.ds(0, GROUP)], r_idx[b], sem_e[b]).wait()
        pltpu.make_async_copy(val_hbm.at[pl.ds(0, GROUP)], vv[b], sem_e[b]).wait()

    def wait_gath(b):
        pltpu.make_async_copy(tab_hbm.at[c_idx[b]], rows[b], sem_g[b]).wait()

    def wait_scat(b):
        pltpu.make_async_copy(rows[b], acc_sh.at[sx[b]], sem_s[b]).wait()

    fire_edge(0, 0)

    def grp2(i, carry):
        for par in range(2):
            g = i * 2 + par
            b = par
            o = 1 - par

            @pl.when(g < cnt)
            def _():
                wait_edge(b)

                @pl.when(g >= 2)
                def _():
                    wait_scat(b)
                pltpu.async_copy(tab_hbm.at[c_idx[b]], rows[b], sem_g[b])

            @pl.when((g >= 1) & (g <= cnt))
            def _():
                wait_gath(o)
                for k in range(GROUP // 16):
                    sl = pl.ds(k * 16, 16)
                    sx[o][sl] = r_idx[o][sl]
                for k in range(GROUP // 16):
                    v16 = vv[o][pl.ds(k * 16, 16)]
                    for m in range(16):
                        e = k * 16 + m
                        ve = v16[m]
                        rows[o][e, pl.ds(0, 16)] = rows[o][e, pl.ds(0, 16)] * ve
                        rows[o][e, pl.ds(16, 16)] = rows[o][e, pl.ds(16, 16)] * ve
                pltpu.async_copy(rows[o], acc_sh.at[sx[o]], sem_s[o], add=True)

            @pl.when(g + 1 < cnt)
            def _():
                fire_edge(g + 1, o)
        return carry
    lax.fori_loop(0, (G_BASE + 2 + 1) // 2, grp2, 0)

    wait_scat(0)
    wait_scat(1)
    plsc.subcore_barrier()
    pltpu.sync_copy(acc_sh.at[pl.ds(row0, rpt)], acc_out.at[c, pl.ds(row0, rpt)])


def _make_ws(rpt):
    nrows = rpt * NTILE
    def run(*args):
        pair = lambda sh, dt: (pltpu.VMEM(sh, dt), pltpu.VMEM(sh, dt))
        sems = lambda: (pltpu.SemaphoreType.DMA, pltpu.SemaphoreType.DMA)
        return pl.kernel(
            functools.partial(_ws_body, rpt),
            out_type=jax.ShapeDtypeStruct((2, nrows, HD), jnp.float32),
            mesh=_mesh(),
            compiler_params=pltpu.CompilerParams(use_tc_tiling_on_sc=False),
            scratch_types=[
                pair((GROUP,), jnp.int32),
                pair((GROUP,), jnp.int32),
                pair((GROUP,), jnp.float32),
                pair((GROUP, HD), jnp.float32),
                pair((GROUP,), jnp.int32),
                pltpu.VMEM_SHARED((nrows, HD), jnp.float32),
                sems(),
                sems(),
                sems(),
            ],
        )(*args)
    return run


_ws_ent = _make_ws(ENT_RPT)
_ws_tot = _make_ws(TOT_RPT)


def _gather_body(ifin_hbm, ufin_hbm, iid_hbm, uid_hbm, irows_out, urows_out,
                 idbuf, rowbuf, sem0):
    c = lax.axis_index("c")
    s = lax.axis_index("s")
    w = s * 2 + c
    base = w * (BATCH // 32)
    n = BATCH // 32
    pltpu.sync_copy(iid_hbm.at[pl.ds(base, n)], idbuf)
    pltpu.async_copy(ifin_hbm.at[idbuf], rowbuf, sem0).wait()
    pltpu.sync_copy(rowbuf, irows_out.at[pl.ds(base, n)])
    pltpu.sync_copy(uid_hbm.at[pl.ds(base, n)], idbuf)
    for k in range(n // 16):
        sl = pl.ds(k * 16, 16)
        idbuf[sl] = idbuf[sl] - N_ENT
    pltpu.async_copy(ufin_hbm.at[idbuf], rowbuf, sem0).wait()
    pltpu.sync_copy(rowbuf, urows_out.at[pl.ds(base, n)])


def _gatherk(*args):
    return pl.kernel(
        _gather_body,
        out_type=[
            jax.ShapeDtypeStruct((BATCH, D), jnp.float32),
            jax.ShapeDtypeStruct((BATCH, D), jnp.float32),
        ],
        mesh=_mesh(),
        compiler_params=pltpu.CompilerParams(use_tc_tiling_on_sc=False),
        scratch_types=[
            pltpu.VMEM((BATCH // 32,), jnp.int32),
            pltpu.VMEM((BATCH // 32, D), jnp.float32),
            pltpu.SemaphoreType.DMA,
        ],
    )(*args)


_RB = 2000


def _ct(a, b):
    return lax.dot_general(a, b, (((1,), (1,)), ((), ())),
                           preferred_element_type=jnp.float32)


def _prep_body(e_ref, rel_ref, wkw_ref, wkb_ref, t32_ref, h32_ref):
    q = jnp.dot(rel_ref[...], wkw_ref[...], preferred_element_type=jnp.float32)
    bias = _ct(wkb_ref[...], rel_ref[...])
    e = e_ref[...]
    t32_ref[...] = _ct(e, q[:, :D])
    h32_ref[...] = _ct(e, q[:, D:]) + bias


def _tc_prep(e_ent, rel, wkw, wkb_row):
    grid = N_ENT // _RB
    return pl.pallas_call(
        _prep_body,
        grid=(grid,),
        in_specs=[
            pl.BlockSpec((_RB, D), lambda i: (i, 0)),
            pl.BlockSpec((NREL, D), lambda i: (0, 0)),
            pl.BlockSpec((D, 2 * D), lambda i: (0, 0)),
            pl.BlockSpec((1, D), lambda i: (0, 0)),
        ],
        out_specs=[
            pl.BlockSpec((_RB, NREL), lambda i: (i, 0)),
            pl.BlockSpec((_RB, NREL), lambda i: (i, 0)),
        ],
        out_shape=[
            jax.ShapeDtypeStruct((N_ENT, NREL), jnp.float32),
            jax.ShapeDtypeStruct((N_ENT, NREL), jnp.float32),
        ],
    )(e_ent, rel, wkw, wkb_row)


def _ent_body(numlo_ref, numhi_ref, den_ref, iglo_ref, ighi_ref, isum_ref,
              rel_ref, wkw_ref, wkb_ref, wa_ref, wb_ref,
              elo_ref, ehi_ref, t32_ref, h32_ref, dlo_ref, dhi_ref, isum_out):
    numlo = numlo_ref[0] + numlo_ref[1]
    numhi = numhi_ref[0] + numhi_ref[1]
    den = den_ref[0, :, 0] + den_ref[1, :, 0]
    inv = (1.0 / (den + 1e-16))[:, None]
    ekg_lo = numlo * inv
    ekg_hi = numhi * inv
    ekg = jnp.concatenate([ekg_lo, ekg_hi], axis=1)
    collab_lo = iglo_ref[0] + iglo_ref[1]
    collab_hi = ighi_ref[0] + ighi_ref[1]
    collab = jnp.concatenate([collab_lo, collab_hi], axis=1)
    g = jax.nn.sigmoid(_ct(ekg, wa_ref[...]) + _ct(collab, wb_ref[...]))
    dual = g * ekg + (1.0 - g) * collab
    q = jnp.dot(rel_ref[...], wkw_ref[...], preferred_element_type=jnp.float32)
    bias = _ct(wkb_ref[...], rel_ref[...])
    elo_ref[...] = ekg_lo
    ehi_ref[...] = ekg_hi
    t32_ref[...] = _ct(ekg, q[:, :D])
    h32_ref[...] = _ct(ekg, q[:, D:]) + bias
    dlo_ref[...] = dual[:, :HD]
    dhi_ref[...] = dual[:, HD:]
    isum_out[...] = isum_ref[...] + collab


def _tc_entity(num_p, numhi_p, den_p, iglo_p, ighi_p, item_sum,
               rel, wkw, wkb_row, wa, wb):
    grid = N_ENT // _RB
    p3 = pl.BlockSpec((2, _RB, HD), lambda i: (0, i, 0))
    full = lambda shape: pl.BlockSpec(shape, lambda i: tuple(0 for _ in shape))
    ob32 = pl.BlockSpec((_RB, HD), lambda i: (i, 0))
    ob64 = pl.BlockSpec((_RB, D), lambda i: (i, 0))
    return pl.pallas_call(
        _ent_body,
        grid=(grid,),
        in_specs=[
            p3, p3,
            pl.BlockSpec((2, _RB, 1), lambda i: (0, i, 0)),
            p3, p3,
            ob64,
            full((NREL, D)), full((D, 2 * D)), full((1, D)),
            full((D, D)), full((D, D)),
        ],
        out_specs=[ob32, ob32, ob32, ob32, ob32, ob32, ob64],
        out_shape=[
            jax.ShapeDtypeStruct((N_ENT, HD), jnp.float32),
            jax.ShapeDtypeStruct((N_ENT, HD), jnp.float32),
            jax.ShapeDtypeStruct((N_ENT, NREL), jnp.float32),
            jax.ShapeDtypeStruct((N_ENT, NREL), jnp.float32),
            jax.ShapeDtypeStruct((N_ENT, HD), jnp.float32),
            jax.ShapeDtypeStruct((N_ENT, HD), jnp.float32),
            jax.ShapeDtypeStruct((N_ENT, D), jnp.float32),
        ],
    )(num_p, numhi_p, den_p, iglo_p, ighi_p, item_sum,
      rel, wkw, wkb_row, wa, wb)


def _usr_body(iglo_ref, ighi_ref, usum_ref, ulo_ref, uhi_ref, usum_out):
    lo = iglo_ref[0] + iglo_ref[1]
    hi = ighi_ref[0] + ighi_ref[1]
    ulo_ref[...] = lo
    uhi_ref[...] = hi
    usum_out[...] = usum_ref[...] + jnp.concatenate([lo, hi], axis=1)


def _tc_user(iglo_p, ighi_p, user_sum):
    grid = N_USR // _RB
    p3 = pl.BlockSpec((2, _RB, HD), lambda i: (0, i, 0))
    ob32 = pl.BlockSpec((_RB, HD), lambda i: (i, 0))
    ob64 = pl.BlockSpec((_RB, D), lambda i: (i, 0))
    return pl.pallas_call(
        _usr_body,
        grid=(grid,),
        in_specs=[p3, p3, ob64],
        out_specs=[ob32, ob32, ob64],
        out_shape=[
            jax.ShapeDtypeStruct((N_USR, HD), jnp.float32),
            jax.ShapeDtypeStruct((N_USR, HD), jnp.float32),
            jax.ShapeDtypeStruct((N_USR, D), jnp.float32),
        ],
    )(iglo_p, ighi_p, user_sum)


def _scores_body(u_ref, i_ref, o_ref):
    o_ref[...] = _ct(u_ref[...], i_ref[...])


def _tc_scores(user_rows, item_rows):
    return pl.pallas_call(
        _scores_body,
        out_shape=jax.ShapeDtypeStruct((BATCH, BATCH), jnp.float32),
    )(user_rows, item_rows)


def kernel(h_list, t_list, r_list, ai_row, ai_col, ai_val, user_ids, item_ids,
           entity_user_embed, relation_embed, Wk_w, Wk_b, Wa, Wb):
    e_ent = entity_user_embed[:N_ENT]
    wkb_row = Wk_b.reshape(1, D)
    T32, H32 = _tc_prep(e_ent, relation_embed, Wk_w, wkb_row)
    e_lo = e_ent[:, :HD]
    e_hi = e_ent[:, HD:]
    ig_lo = entity_user_embed[:, :HD]
    ig_hi = entity_user_embed[:, HD:]
    item_sum = e_ent
    user_sum = entity_user_embed[N_ENT:]
    for _ in range(2):
        w_e, numlo_p, den_p0, den_p1 = _kg1(h_list, t_list, r_list,
                                            T32.reshape(-1), H32.reshape(-1),
                                            e_lo)
        den_p = jnp.stack([den_p0[:N_ENT], den_p1[:N_ENT]]).reshape(2, N_ENT, 1)
        numhi_p = _ws_ent(t_list, h_list, w_e, e_hi)
        iglo_p = _ws_tot(ai_col, ai_row, ai_val, ig_lo)
        ighi_p = _ws_tot(ai_col, ai_row, ai_val, ig_hi)
        e_lo, e_hi, T32, H32, d_lo, d_hi, item_sum = _tc_entity(
            numlo_p[:, :N_ENT], numhi_p[:, :N_ENT], den_p,
            iglo_p[:, :N_ENT], ighi_p[:, :N_ENT], item_sum,
            relation_embed, Wk_w, wkb_row, Wa, Wb)
        u_lo, u_hi, user_sum = _tc_user(
            iglo_p[:, N_ENT:N_TOT], ighi_p[:, N_ENT:N_TOT], user_sum)
        ig_lo = jnp.concatenate([d_lo, u_lo], axis=0)
        ig_hi = jnp.concatenate([d_hi, u_hi], axis=0)
    item_rows, user_rows = _gatherk(item_sum, user_sum, item_ids, user_ids)
    return _tc_scores(user_rows, item_rows)


# speedup vs baseline: 1.0006x; 1.0006x over previous
import functools

import jax
import jax.numpy as jnp
from jax import lax
from jax.experimental import pallas as pl
from jax.experimental.pallas import tpu as pltpu
import jax.experimental.pallas.tpu_sc as plsc

N_ENT = 50000
N_USR = 10000
N_TOT = 60000
D = 64
HD = 32
NREL = 32
E = 800000
BATCH = 1024

GROUP = 128
G_TOTAL = E // GROUP
G_PER_SC = G_TOTAL // 2
NTILE = 16
G_BASE = G_PER_SC // NTILE
G_REM = G_PER_SC % NTILE
ZROWS = 104
ZDEN = 520
ENT_RPT = 3128
TOT_RPT = 3752
N_PENT = ENT_RPT * NTILE
N_PTOT = TOT_RPT * NTILE
DEN_RPT = ENT_RPT
N_DEN = N_PENT

_mesh = lambda: plsc.VectorSubcoreMesh(core_axis_name="c", subcore_axis_name="s")


def _zero_z2d(z2d):
    def zb(i, carry):
        z2d[i, pl.ds(0, 16)] = jnp.zeros((16,), jnp.float32)
        z2d[i, pl.ds(16, 16)] = jnp.zeros((16,), jnp.float32)
        return carry
    lax.fori_loop(0, ZROWS, zb, 0)


def _zero_acc(z2d, acc_sh, row0, rpt):
    nz = rpt // ZROWS
    def zs(i, carry):
        pltpu.sync_copy(z2d, acc_sh.at[pl.ds(row0 + i * ZROWS, ZROWS)])
        return carry
    lax.fori_loop(0, nz, zs, 0)
    pltpu.sync_copy(z2d.at[pl.ds(0, 8)], acc_sh.at[pl.ds(row0 + rpt - 8, 8)])


def _group_span(c, s):
    lo_t = s * G_BASE + jnp.minimum(s, G_REM)
    cnt = G_BASE + jnp.where(s < G_REM, 1, 0)
    g0 = c * G_PER_SC + lo_t
    return g0, cnt


def _kg1_body(h_hbm, t_hbm, r_hbm, tf_hbm, hf_hbm, elo_hbm,
              w_hbm, num_out, den_out0, den_out1,
              h_idx, t_idx, r_idx, ti, hi2, av, bv, wv, rows, sx,
              num_sh, den_sh, sem_e, sem_g, sem_s, sem_w):
    c = lax.axis_index("c")
    s = lax.axis_index("s")
    row0 = s * ENT_RPT

    z16 = jnp.zeros((16,), jnp.float32)
    def zr(i, carry):
        rows[0][i, pl.ds(0, 16)] = z16
        rows[0][i, pl.ds(16, 16)] = z16
        return carry
    lax.fori_loop(0, GROUP, zr, 0)
    for k in range(GROUP // 16):
        wv[0][pl.ds(k * 16, 16)] = z16

    def za(i, carry):
        pltpu.sync_copy(rows[0], num_sh.at[pl.ds(row0 + i * GROUP, GROUP)])
        return carry
    lax.fori_loop(0, ENT_RPT // GROUP, za, 0)
    pltpu.sync_copy(rows[0], num_sh.at[pl.ds(row0 + ENT_RPT - GROUP, GROUP)])

    def zd(i, carry):
        pltpu.sync_copy(wv[0], den_sh.at[pl.ds(row0 + i * GROUP, GROUP)])
        return carry
    lax.fori_loop(0, DEN_RPT // GROUP, zd, 0)
    pltpu.sync_copy(wv[0], den_sh.at[pl.ds(row0 + DEN_RPT - GROUP, GROUP)])
    plsc.subcore_barrier()

    g0, cnt = _group_span(c, s)

    def fire_edge(g, b):
        base = (g0 + g) * GROUP
        pltpu.async_copy(h_hbm.at[pl.ds(base, GROUP)], h_idx[b], sem_e[b])
        pltpu.async_copy(t_hbm.at[pl.ds(base, GROUP)], t_idx[b], sem_e[b])
        pltpu.async_copy(r_hbm.at[pl.ds(base, GROUP)], r_idx[b], sem_e[b])

    def wait_edge(b):
        pltpu.make_async_copy(h_hbm.at[pl.ds(0, GROUP)], h_idx[b], sem_e[b]).wait()
        pltpu.make_async_copy(h_hbm.at[pl.ds(0, GROUP)], t_idx[b], sem_e[b]).wait()
        pltpu.make_async_copy(h_hbm.at[pl.ds(0, GROUP)], r_idx[b], sem_e[b]).wait()

    def wait_gath(b):
        pltpu.make_async_copy(tf_hbm.at[ti[b]], av[b], sem_g[b]).wait()
        pltpu.make_async_copy(hf_hbm.at[hi2[b]], bv[b], sem_g[b]).wait()
        pltpu.make_async_copy(elo_hbm.at[t_idx[b]], rows[b], sem_g[b]).wait()

    def wait_scat(b):
        pltpu.make_async_copy(rows[b], num_sh.at[sx[b]], sem_s[b]).wait()
        pltpu.make_async_copy(wv[b], den_sh.at[sx[b]], sem_s[b]).wait()
        pltpu.make_async_copy(wv[b], w_hbm.at[pl.ds(0, GROUP)], sem_w[b]).wait()

    fire_edge(0, 0)
    wait_edge(0)
    for k in range(GROUP // 16):
        sl = pl.ds(k * 16, 16)
        rr = r_idx[0][sl]
        ti[0][sl] = t_idx[0][sl] * NREL + rr
        hi2[0][sl] = h_idx[0][sl] * NREL + rr
    pltpu.async_copy(tf_hbm.at[ti[0]], av[0], sem_g[0])
    pltpu.async_copy(hf_hbm.at[hi2[0]], bv[0], sem_g[0])
    pltpu.async_copy(elo_hbm.at[t_idx[0]], rows[0], sem_g[0])
    fire_edge(1, 1)

    def grp2(i, carry):
        for par in range(2):
            g = i * 2 + par
            b = par
            o = 1 - par

            @pl.when(g + 1 < cnt)
            def _():
                wait_edge(o)
                for k in range(GROUP // 16):
                    sl = pl.ds(k * 16, 16)
                    rr = r_idx[o][sl]
                    ti[o][sl] = t_idx[o][sl] * NREL + rr
                    hi2[o][sl] = h_idx[o][sl] * NREL + rr

                @pl.when(g >= 1)
                def _():
                    wait_scat(o)
                pltpu.async_copy(tf_hbm.at[ti[o]], av[o], sem_g[o])
                pltpu.async_copy(hf_hbm.at[hi2[o]], bv[o], sem_g[o])
                pltpu.async_copy(elo_hbm.at[t_idx[o]], rows[o], sem_g[o])

            @pl.when(g < cnt)
            def _():
                wait_gath(b)
                for k in range(GROUP // 16):
                    sl = pl.ds(k * 16, 16)
                    v = av[b][sl] + bv[b][sl]
                    v = jnp.maximum(v, v * 0.01)
                    wv[b][sl] = jnp.exp(v)
                    sx[b][sl] = h_idx[b][sl]
                for k in range(GROUP // 16):
                    w16 = wv[b][pl.ds(k * 16, 16)]
                    for m in range(16):
                        e = k * 16 + m
                        we = w16[m]
                        rows[b][e, pl.ds(0, 16)] = rows[b][e, pl.ds(0, 16)] * we
                        rows[b][e, pl.ds(16, 16)] = rows[b][e, pl.ds(16, 16)] * we
                base_g = (g0 + g) * GROUP
                pltpu.async_copy(rows[b], num_sh.at[sx[b]], sem_s[b], add=True)
                pltpu.async_copy(wv[b], den_sh.at[sx[b]], sem_s[b], add=True)
                pltpu.async_copy(wv[b], w_hbm.at[pl.ds(base_g, GROUP)],
                                 sem_w[b])

            @pl.when(g + 2 < cnt)
            def _():
                fire_edge(g + 2, b)
        return carry
    lax.fori_loop(0, (G_BASE + 2) // 2, grp2, 0)

    wait_scat(0)
    wait_scat(1)
    plsc.subcore_barrier()
    pltpu.sync_copy(num_sh.at[pl.ds(row0, ENT_RPT)],
                    num_out.at[c, pl.ds(row0, ENT_RPT)])

    @pl.when(c == 0)
    def _():
        pltpu.sync_copy(den_sh.at[pl.ds(row0, DEN_RPT)],
                        den_out0.at[pl.ds(row0, DEN_RPT)])

    @pl.when(c == 1)
    def _():
        pltpu.sync_copy(den_sh.at[pl.ds(row0, DEN_RPT)],
                        den_out1.at[pl.ds(row0, DEN_RPT)])


def _kg1(*args):
    pair = lambda sh, dt: (pltpu.VMEM(sh, dt), pltpu.VMEM(sh, dt))
    sems = lambda: (pltpu.SemaphoreType.DMA, pltpu.SemaphoreType.DMA)
    return pl.kernel(
        _kg1_body,
        out_type=[
            jax.ShapeDtypeStruct((E,), jnp.float32),
            jax.ShapeDtypeStruct((2, N_PENT, HD), jnp.float32),
            jax.ShapeDtypeStruct((N_DEN,), jnp.float32),
            jax.ShapeDtypeStruct((N_DEN,), jnp.float32),
        ],
        mesh=_mesh(),
        compiler_params=pltpu.CompilerParams(use_tc_tiling_on_sc=False),
        scratch_types=[
            pair((GROUP,), jnp.int32),
            pair((GROUP,), jnp.int32),
            pair((GROUP,), jnp.int32),
            pair((GROUP,), jnp.int32),
            pair((GROUP,), jnp.int32),
            pair((GROUP,), jnp.float32),
            pair((GROUP,), jnp.float32),
            pair((GROUP,), jnp.float32),
            pair((GROUP, HD), jnp.float32),
            pair((GROUP,), jnp.int32),
            pltpu.VMEM_SHARED((N_PENT, HD), jnp.float32),
            pltpu.VMEM_SHARED((N_DEN,), jnp.float32),
            sems(),
            sems(),
            sems(),
            sems(),
        ],
    )(*args)


def _ws_body(rpt, col_hbm, row_hbm, val_hbm, tab_hbm, acc_out,
             c_idx, r_idx, vv, rows, sx, acc_sh, sem_e, sem_g, sem_s):
    c = lax.axis_index("c")
    s = lax.axis_index("s")
    row0 = s * rpt

    z16 = jnp.zeros((16,), jnp.float32)
    def zr(i, carry):
        rows[0][i, pl.ds(0, 16)] = z16
        rows[0][i, pl.ds(16, 16)] = z16
        return carry
    lax.fori_loop(0, GROUP, zr, 0)

    def za(i, carry):
        pltpu.sync_copy(rows[0], acc_sh.at[pl.ds(row0 + i * GROUP, GROUP)])
        return carry
    lax.fori_loop(0, rpt // GROUP, za, 0)
    pltpu.sync_copy(rows[0], acc_sh.at[pl.ds(row0 + rpt - GROUP, GROUP)])
    plsc.subcore_barrier()

    g0, cnt = _group_span(c, s)

    def fire_edge(g, b):
        base = (g0 + g) * GROUP
        pltpu.async_copy(col_hbm.at[pl.ds(base, GROUP)], c_idx[b], sem_e[b])
        pltpu.async_copy(row_hbm.at[pl.ds(base, GROUP)], r_idx[b], sem_e[b])
        pltpu.async_copy(val_hbm.at[pl.ds(base, GROUP)], vv[b], sem_e[b])

    def wait_edge(b):
        pltpu.make_async_copy(col_hbm.at[pl.ds(0, GROUP)], c_idx[b], sem_e[b]).wait()
        pltpu.make_async_copy(col_hbm.at[pl.ds(0, GROUP)], r_idx[b], sem_e[b]).wait()
        pltpu.make_async_copy(val_hbm.at[pl.ds(0, GROUP)], vv[b], sem_e[b]).wait()

    def wait_gath(b):
        pltpu.make_async_copy(tab_hbm.at[c_idx[b]], rows[b], sem_g[b]).wait()

    def wait_scat(b):
        pltpu.make_async_copy(rows[b], acc_sh.at[sx[b]], sem_s[b]).wait()

    fire_edge(0, 0)
    wait_edge(0)
    pltpu.async_copy(tab_hbm.at[c_idx[0]], rows[0], sem_g[0])
    fire_edge(1, 1)

    def grp2(i, carry):
        for par in range(2):
            g = i * 2 + par
            b = par
            o = 1 - par

            @pl.when(g + 1 < cnt)
            def _():
                wait_edge(o)

                @pl.when(g >= 1)
                def _():
                    wait_scat(o)
                pltpu.async_copy(tab_hbm.at[c_idx[o]], rows[o], sem_g[o])

            @pl.when(g < cnt)
            def _():
                wait_gath(b)
                for k in range(GROUP // 16):
                    sl = pl.ds(k * 16, 16)
                    sx[b][sl] = r_idx[b][sl]
                for k in range(GROUP // 16):
                    v16 = vv[b][pl.ds(k * 16, 16)]
                    for m in range(16):
                        e = k * 16 + m
                        ve = v16[m]
                        rows[b][e, pl.ds(0, 16)] = rows[b][e, pl.ds(0, 16)] * ve
                        rows[b][e, pl.ds(16, 16)] = rows[b][e, pl.ds(16, 16)] * ve
                pltpu.async_copy(rows[b], acc_sh.at[sx[b]], sem_s[b], add=True)

            @pl.when(g + 2 < cnt)
            def _():
                fire_edge(g + 2, b)
        return carry
    lax.fori_loop(0, (G_BASE + 2) // 2, grp2, 0)

    wait_scat(0)
    wait_scat(1)
    plsc.subcore_barrier()
    pltpu.sync_copy(acc_sh.at[pl.ds(row0, rpt)], acc_out.at[c, pl.ds(row0, rpt)])


def _make_ws(rpt):
    nrows = rpt * NTILE
    def run(*args):
        pair = lambda sh, dt: (pltpu.VMEM(sh, dt), pltpu.VMEM(sh, dt))
        sems = lambda: (pltpu.SemaphoreType.DMA, pltpu.SemaphoreType.DMA)
        return pl.kernel(
            functools.partial(_ws_body, rpt),
            out_type=jax.ShapeDtypeStruct((2, nrows, HD), jnp.float32),
            mesh=_mesh(),
            compiler_params=pltpu.CompilerParams(use_tc_tiling_on_sc=False),
            scratch_types=[
                pair((GROUP,), jnp.int32),
                pair((GROUP,), jnp.int32),
                pair((GROUP,), jnp.float32),
                pair((GROUP, HD), jnp.float32),
                pair((GROUP,), jnp.int32),
                pltpu.VMEM_SHARED((nrows, HD), jnp.float32),
                sems(),
                sems(),
                sems(),
            ],
        )(*args)
    return run


_ws_ent = _make_ws(ENT_RPT)
_ws_tot = _make_ws(TOT_RPT)


def _gather_body(ifin_hbm, ufin_hbm, iid_hbm, uid_hbm, irows_out, urows_out,
                 idbuf, rowbuf, sem0):
    c = lax.axis_index("c")
    s = lax.axis_index("s")
    w = s * 2 + c
    base = w * (BATCH // 32)
    n = BATCH // 32
    pltpu.sync_copy(iid_hbm.at[pl.ds(base, n)], idbuf)
    pltpu.async_copy(ifin_hbm.at[idbuf], rowbuf, sem0).wait()
    pltpu.sync_copy(rowbuf, irows_out.at[pl.ds(base, n)])
    pltpu.sync_copy(uid_hbm.at[pl.ds(base, n)], idbuf)
    for k in range(n // 16):
        sl = pl.ds(k * 16, 16)
        idbuf[sl] = idbuf[sl] - N_ENT
    pltpu.async_copy(ufin_hbm.at[idbuf], rowbuf, sem0).wait()
    pltpu.sync_copy(rowbuf, urows_out.at[pl.ds(base, n)])


def _gatherk(*args):
    return pl.kernel(
        _gather_body,
        out_type=[
            jax.ShapeDtypeStruct((BATCH, D), jnp.float32),
            jax.ShapeDtypeStruct((BATCH, D), jnp.float32),
        ],
        mesh=_mesh(),
        compiler_params=pltpu.CompilerParams(use_tc_tiling_on_sc=False),
        scratch_types=[
            pltpu.VMEM((BATCH // 32,), jnp.int32),
            pltpu.VMEM((BATCH // 32, D), jnp.float32),
            pltpu.SemaphoreType.DMA,
        ],
    )(*args)


_RB = 2000


def _ct(a, b):
    return lax.dot_general(a, b, (((1,), (1,)), ((), ())),
                           preferred_element_type=jnp.float32)


def _prep_body(e_ref, rel_ref, wkw_ref, wkb_ref, t32_ref, h32_ref):
    q = jnp.dot(rel_ref[...], wkw_ref[...], preferred_element_type=jnp.float32)
    bias = _ct(wkb_ref[...], rel_ref[...])
    e = e_ref[...]
    t32_ref[...] = _ct(e, q[:, :D])
    h32_ref[...] = _ct(e, q[:, D:]) + bias


def _tc_prep(e_ent, rel, wkw, wkb_row):
    grid = N_ENT // _RB
    return pl.pallas_call(
        _prep_body,
        grid=(grid,),
        in_specs=[
            pl.BlockSpec((_RB, D), lambda i: (i, 0)),
            pl.BlockSpec((NREL, D), lambda i: (0, 0)),
            pl.BlockSpec((D, 2 * D), lambda i: (0, 0)),
            pl.BlockSpec((1, D), lambda i: (0, 0)),
        ],
        out_specs=[
            pl.BlockSpec((_RB, NREL), lambda i: (i, 0)),
            pl.BlockSpec((_RB, NREL), lambda i: (i, 0)),
        ],
        out_shape=[
            jax.ShapeDtypeStruct((N_ENT, NREL), jnp.float32),
            jax.ShapeDtypeStruct((N_ENT, NREL), jnp.float32),
        ],
    )(e_ent, rel, wkw, wkb_row)


def _ent_body(numlo_ref, numhi_ref, den_ref, iglo_ref, ighi_ref, isum_ref,
              rel_ref, wkw_ref, wkb_ref, wa_ref, wb_ref,
              elo_ref, ehi_ref, t32_ref, h32_ref, dlo_ref, dhi_ref, isum_out):
    numlo = numlo_ref[0] + numlo_ref[1]
    numhi = numhi_ref[0] + numhi_ref[1]
    den = den_ref[0, :, 0] + den_ref[1, :, 0]
    inv = (1.0 / (den + 1e-16))[:, None]
    ekg_lo = numlo * inv
    ekg_hi = numhi * inv
    ekg = jnp.concatenate([ekg_lo, ekg_hi], axis=1)
    collab_lo = iglo_ref[0] + iglo_ref[1]
    collab_hi = ighi_ref[0] + ighi_ref[1]
    collab = jnp.concatenate([collab_lo, collab_hi], axis=1)
    g = jax.nn.sigmoid(_ct(ekg, wa_ref[...]) + _ct(collab, wb_ref[...]))
    dual = g * ekg + (1.0 - g) * collab
    q = jnp.dot(rel_ref[...], wkw_ref[...], preferred_element_type=jnp.float32)
    bias = _ct(wkb_ref[...], rel_ref[...])
    elo_ref[...] = ekg_lo
    ehi_ref[...] = ekg_hi
    t32_ref[...] = _ct(ekg, q[:, :D])
    h32_ref[...] = _ct(ekg, q[:, D:]) + bias
    dlo_ref[...] = dual[:, :HD]
    dhi_ref[...] = dual[:, HD:]
    isum_out[...] = isum_ref[...] + collab


def _tc_entity(num_p, numhi_p, den_p, iglo_p, ighi_p, item_sum,
               rel, wkw, wkb_row, wa, wb):
    grid = N_ENT // _RB
    p3 = pl.BlockSpec((2, _RB, HD), lambda i: (0, i, 0))
    full = lambda shape: pl.BlockSpec(shape, lambda i: tuple(0 for _ in shape))
    ob32 = pl.BlockSpec((_RB, HD), lambda i: (i, 0))
    ob64 = pl.BlockSpec((_RB, D), lambda i: (i, 0))
    return pl.pallas_call(
        _ent_body,
        grid=(grid,),
        in_specs=[
            p3, p3,
            pl.BlockSpec((2, _RB, 1), lambda i: (0, i, 0)),
            p3, p3,
            ob64,
            full((NREL, D)), full((D, 2 * D)), full((1, D)),
            full((D, D)), full((D, D)),
        ],
        out_specs=[ob32, ob32, ob32, ob32, ob32, ob32, ob64],
        out_shape=[
            jax.ShapeDtypeStruct((N_ENT, HD), jnp.float32),
            jax.ShapeDtypeStruct((N_ENT, HD), jnp.float32),
            jax.ShapeDtypeStruct((N_ENT, NREL), jnp.float32),
            jax.ShapeDtypeStruct((N_ENT, NREL), jnp.float32),
            jax.ShapeDtypeStruct((N_ENT, HD), jnp.float32),
            jax.ShapeDtypeStruct((N_ENT, HD), jnp.float32),
            jax.ShapeDtypeStruct((N_ENT, D), jnp.float32),
        ],
    )(num_p, numhi_p, den_p, iglo_p, ighi_p, item_sum,
      rel, wkw, wkb_row, wa, wb)


def _usr_body(iglo_ref, ighi_ref, usum_ref, ulo_ref, uhi_ref, usum_out):
    lo = iglo_ref[0] + iglo_ref[1]
    hi = ighi_ref[0] + ighi_ref[1]
    ulo_ref[...] = lo
    uhi_ref[...] = hi
    usum_out[...] = usum_ref[...] + jnp.concatenate([lo, hi], axis=1)


def _tc_user(iglo_p, ighi_p, user_sum):
    grid = N_USR // _RB
    p3 = pl.BlockSpec((2, _RB, HD), lambda i: (0, i, 0))
    ob32 = pl.BlockSpec((_RB, HD), lambda i: (i, 0))
    ob64 = pl.BlockSpec((_RB, D), lambda i: (i, 0))
    return pl.pallas_call(
        _usr_body,
        grid=(grid,),
        in_specs=[p3, p3, ob64],
        out_specs=[ob32, ob32, ob64],
        out_shape=[
            jax.ShapeDtypeStruct((N_USR, HD), jnp.float32),
            jax.ShapeDtypeStruct((N_USR, HD), jnp.float32),
            jax.ShapeDtypeStruct((N_USR, D), jnp.float32),
        ],
    )(iglo_p, ighi_p, user_sum)


def _scores_body(u_ref, i_ref, o_ref):
    o_ref[...] = _ct(u_ref[...], i_ref[...])


def _tc_scores(user_rows, item_rows):
    return pl.pallas_call(
        _scores_body,
        out_shape=jax.ShapeDtypeStruct((BATCH, BATCH), jnp.float32),
    )(user_rows, item_rows)


def kernel(h_list, t_list, r_list, ai_row, ai_col, ai_val, user_ids, item_ids,
           entity_user_embed, relation_embed, Wk_w, Wk_b, Wa, Wb):
    e_ent = entity_user_embed[:N_ENT]
    wkb_row = Wk_b.reshape(1, D)
    T32, H32 = _tc_prep(e_ent, relation_embed, Wk_w, wkb_row)
    e_lo = e_ent[:, :HD]
    e_hi = e_ent[:, HD:]
    ig_lo = entity_user_embed[:, :HD]
    ig_hi = entity_user_embed[:, HD:]
    item_sum = e_ent
    user_sum = entity_user_embed[N_ENT:]
    for _ in range(2):
        w_e, numlo_p, den_p0, den_p1 = _kg1(h_list, t_list, r_list,
                                            T32.reshape(-1), H32.reshape(-1),
                                            e_lo)
        den_p = jnp.stack([den_p0[:N_ENT], den_p1[:N_ENT]]).reshape(2, N_ENT, 1)
        numhi_p = _ws_ent(t_list, h_list, w_e, e_hi)
        iglo_p = _ws_tot(ai_col, ai_row, ai_val, ig_lo)
        ighi_p = _ws_tot(ai_col, ai_row, ai_val, ig_hi)
        e_lo, e_hi, T32, H32, d_lo, d_hi, item_sum = _tc_entity(
            numlo_p[:, :N_ENT], numhi_p[:, :N_ENT], den_p,
            iglo_p[:, :N_ENT], ighi_p[:, :N_ENT], item_sum,
            relation_embed, Wk_w, wkb_row, Wa, Wb)
        u_lo, u_hi, user_sum = _tc_user(
            iglo_p[:, N_ENT:N_TOT], ighi_p[:, N_ENT:N_TOT], user_sum)
        ig_lo = jnp.concatenate([d_lo, u_lo], axis=0)
        ig_hi = jnp.concatenate([d_hi, u_hi], axis=0)
    item_rows, user_rows = _gatherk(item_sum, user_sum, item_ids, user_ids)
    return _tc_scores(user_rows, item_rows)


# speedup vs baseline: 1.1209x; 1.1202x over previous
import functools

import jax
import jax.numpy as jnp
from jax import lax
from jax.experimental import pallas as pl
from jax.experimental.pallas import tpu as pltpu
import jax.experimental.pallas.tpu_sc as plsc

N_ENT = 50000
N_USR = 10000
N_TOT = 60000
D = 64
HD = 32
NREL = 32
E = 800000
BATCH = 1024

GROUP = 128
G_TOTAL = E // GROUP
G_PER_SC = G_TOTAL // 2
NTILE = 16
G_BASE = G_PER_SC // NTILE
G_REM = G_PER_SC % NTILE
ZROWS = 104
ZDEN = 520
ENT_RPT = 3128
TOT_RPT = 3752
N_PENT = ENT_RPT * NTILE
N_PTOT = TOT_RPT * NTILE
DEN_RPT = ENT_RPT
N_DEN = N_PENT

_mesh = lambda: plsc.VectorSubcoreMesh(core_axis_name="c", subcore_axis_name="s")


def _zero_z2d(z2d):
    def zb(i, carry):
        z2d[i, pl.ds(0, 16)] = jnp.zeros((16,), jnp.float32)
        z2d[i, pl.ds(16, 16)] = jnp.zeros((16,), jnp.float32)
        return carry
    lax.fori_loop(0, ZROWS, zb, 0)


def _zero_acc(z2d, acc_sh, row0, rpt):
    nz = rpt // ZROWS
    def zs(i, carry):
        pltpu.sync_copy(z2d, acc_sh.at[pl.ds(row0 + i * ZROWS, ZROWS)])
        return carry
    lax.fori_loop(0, nz, zs, 0)
    pltpu.sync_copy(z2d.at[pl.ds(0, 8)], acc_sh.at[pl.ds(row0 + rpt - 8, 8)])


def _group_span(c, s):
    lo_t = s * G_BASE + jnp.minimum(s, G_REM)
    cnt = G_BASE + jnp.where(s < G_REM, 1, 0)
    g0 = c * G_PER_SC + lo_t
    return g0, cnt


def _kg1_body(h_hbm, t_hbm, r_hbm, tf_hbm, hf_hbm, elo_hbm,
              w_hbm, num_out, den_out0, den_out1,
              h_idx, t_idx, r_idx, ti, hi2, av, bv, wv, rows, sx,
              num_sh, den_sh, sem_e, sem_g, sem_s, sem_w):
    c = lax.axis_index("c")
    s = lax.axis_index("s")
    row0 = s * ENT_RPT

    z16 = jnp.zeros((16,), jnp.float32)
    def zr(i, carry):
        rows[0][i, pl.ds(0, 16)] = z16
        rows[0][i, pl.ds(16, 16)] = z16
        return carry
    lax.fori_loop(0, GROUP, zr, 0)
    for k in range(GROUP // 16):
        wv[0][pl.ds(k * 16, 16)] = z16

    def za(i, carry):
        pltpu.sync_copy(rows[0], num_sh.at[pl.ds(row0 + i * GROUP, GROUP)])
        return carry
    lax.fori_loop(0, ENT_RPT // GROUP, za, 0)
    pltpu.sync_copy(rows[0], num_sh.at[pl.ds(row0 + ENT_RPT - GROUP, GROUP)])

    def zd(i, carry):
        pltpu.sync_copy(wv[0], den_sh.at[pl.ds(row0 + i * GROUP, GROUP)])
        return carry
    lax.fori_loop(0, DEN_RPT // GROUP, zd, 0)
    pltpu.sync_copy(wv[0], den_sh.at[pl.ds(row0 + DEN_RPT - GROUP, GROUP)])
    plsc.subcore_barrier()

    g0, cnt = _group_span(c, s)

    def fire_edge(g, b):
        base = (g0 + g) * GROUP
        pltpu.async_copy(h_hbm.at[pl.ds(base, GROUP)], h_idx[b], sem_e[b])
        pltpu.async_copy(t_hbm.at[pl.ds(base, GROUP)], t_idx[b], sem_e[b])
        pltpu.async_copy(r_hbm.at[pl.ds(base, GROUP)], r_idx[b], sem_e[b])

    def wait_edge(b):
        pltpu.make_async_copy(h_hbm.at[pl.ds(0, GROUP)], h_idx[b], sem_e[b]).wait()
        pltpu.make_async_copy(h_hbm.at[pl.ds(0, GROUP)], t_idx[b], sem_e[b]).wait()
        pltpu.make_async_copy(h_hbm.at[pl.ds(0, GROUP)], r_idx[b], sem_e[b]).wait()

    def wait_gath(b):
        pltpu.make_async_copy(tf_hbm.at[ti[b]], av[b], sem_g[b]).wait()
        pltpu.make_async_copy(hf_hbm.at[hi2[b]], bv[b], sem_g[b]).wait()
        pltpu.make_async_copy(elo_hbm.at[t_idx[b]], rows[b], sem_g[b]).wait()

    def wait_scat(b):
        pltpu.make_async_copy(rows[b], num_sh.at[sx[b]], sem_s[b]).wait()
        pltpu.make_async_copy(wv[b], den_sh.at[sx[b]], sem_s[b]).wait()
        pltpu.make_async_copy(wv[b], w_hbm.at[pl.ds(0, GROUP)], sem_w[b]).wait()

    fire_edge(0, 0)
    wait_edge(0)
    for k in range(GROUP // 16):
        sl = pl.ds(k * 16, 16)
        rr = r_idx[0][sl]
        ti[0][sl] = t_idx[0][sl] * NREL + rr
        hi2[0][sl] = h_idx[0][sl] * NREL + rr
    pltpu.async_copy(tf_hbm.at[ti[0]], av[0], sem_g[0])
    pltpu.async_copy(hf_hbm.at[hi2[0]], bv[0], sem_g[0])
    pltpu.async_copy(elo_hbm.at[t_idx[0]], rows[0], sem_g[0])
    fire_edge(1, 1)

    def grp2(i, carry):
        for par in range(2):
            g = i * 2 + par
            b = par
            o = 1 - par

            @pl.when(g + 1 < cnt)
            def _():
                wait_edge(o)
                for k in range(GROUP // 16):
                    sl = pl.ds(k * 16, 16)
                    rr = r_idx[o][sl]
                    ti[o][sl] = t_idx[o][sl] * NREL + rr
                    hi2[o][sl] = h_idx[o][sl] * NREL + rr

                @pl.when(g >= 1)
                def _():
                    wait_scat(o)
                pltpu.async_copy(tf_hbm.at[ti[o]], av[o], sem_g[o])
                pltpu.async_copy(hf_hbm.at[hi2[o]], bv[o], sem_g[o])
                pltpu.async_copy(elo_hbm.at[t_idx[o]], rows[o], sem_g[o])

            @pl.when(g < cnt)
            def _():
                wait_gath(b)
                for k in range(GROUP // 16):
                    sl = pl.ds(k * 16, 16)
                    v = av[b][sl] + bv[b][sl]
                    v = jnp.maximum(v, v * 0.01)
                    wv[b][sl] = jnp.exp(v)
                    sx[b][sl] = h_idx[b][sl]
                for k in range(GROUP // 16):
                    w16 = wv[b][pl.ds(k * 16, 16)]
                    for m in range(16):
                        e = k * 16 + m
                        we = w16[m]
                        rows[b][e, pl.ds(0, 16)] = rows[b][e, pl.ds(0, 16)] * we
                        rows[b][e, pl.ds(16, 16)] = rows[b][e, pl.ds(16, 16)] * we
                base_g = (g0 + g) * GROUP
                pltpu.async_copy(rows[b], num_sh.at[sx[b]], sem_s[b], add=True)
                pltpu.async_copy(wv[b], den_sh.at[sx[b]], sem_s[b], add=True)
                pltpu.async_copy(wv[b], w_hbm.at[pl.ds(base_g, GROUP)],
                                 sem_w[b])

            @pl.when(g + 2 < cnt)
            def _():
                fire_edge(g + 2, b)
        return carry
    lax.fori_loop(0, (G_BASE + 2) // 2, grp2, 0)

    wait_scat(0)
    wait_scat(1)
    plsc.subcore_barrier()
    pltpu.sync_copy(num_sh.at[pl.ds(row0, ENT_RPT)],
                    num_out.at[c, pl.ds(row0, ENT_RPT)])

    @pl.when(c == 0)
    def _():
        pltpu.sync_copy(den_sh.at[pl.ds(row0, DEN_RPT)],
                        den_out0.at[pl.ds(row0, DEN_RPT)])

    @pl.when(c == 1)
    def _():
        pltpu.sync_copy(den_sh.at[pl.ds(row0, DEN_RPT)],
                        den_out1.at[pl.ds(row0, DEN_RPT)])


def _kg1(*args):
    pair = lambda sh, dt: (pltpu.VMEM(sh, dt), pltpu.VMEM(sh, dt))
    sems = lambda: (pltpu.SemaphoreType.DMA, pltpu.SemaphoreType.DMA)
    return pl.kernel(
        _kg1_body,
        out_type=[
            jax.ShapeDtypeStruct((E,), jnp.float32),
            jax.ShapeDtypeStruct((2, N_PENT, HD), jnp.float32),
            jax.ShapeDtypeStruct((N_DEN,), jnp.float32),
            jax.ShapeDtypeStruct((N_DEN,), jnp.float32),
        ],
        mesh=_mesh(),
        compiler_params=pltpu.CompilerParams(use_tc_tiling_on_sc=False),
        scratch_types=[
            pair((GROUP,), jnp.int32),
            pair((GROUP,), jnp.int32),
            pair((GROUP,), jnp.int32),
            pair((GROUP,), jnp.int32),
            pair((GROUP,), jnp.int32),
            pair((GROUP,), jnp.float32),
            pair((GROUP,), jnp.float32),
            pair((GROUP,), jnp.float32),
            pair((GROUP, HD), jnp.float32),
            pair((GROUP,), jnp.int32),
            pltpu.VMEM_SHARED((N_PENT, HD), jnp.float32),
            pltpu.VMEM_SHARED((N_DEN,), jnp.float32),
            sems(),
            sems(),
            sems(),
            sems(),
        ],
    )(*args)


def _ws_body(rpt, col_hbm, row_hbm, val_hbm, tab_hbm, acc_out,
             c_idx, r_idx, vv, rows, sx, acc_sh, sem_e, sem_g, sem_s):
    c = lax.axis_index("c")
    s = lax.axis_index("s")
    row0 = s * rpt

    z16 = jnp.zeros((16,), jnp.float32)
    def zr(i, carry):
        rows[0][i, pl.ds(0, 16)] = z16
        rows[0][i, pl.ds(16, 16)] = z16
        return carry
    lax.fori_loop(0, GROUP, zr, 0)

    def za(i, carry):
        pltpu.sync_copy(rows[0], acc_sh.at[pl.ds(row0 + i * GROUP, GROUP)])
        return carry
    lax.fori_loop(0, rpt // GROUP, za, 0)
    pltpu.sync_copy(rows[0], acc_sh.at[pl.ds(row0 + rpt - GROUP, GROUP)])
    plsc.subcore_barrier()

    g0, cnt = _group_span(c, s)

    def fire_edge(g, b):
        base = (g0 + g) * GROUP
        pltpu.async_copy(col_hbm.at[pl.ds(base, GROUP)], c_idx[b], sem_e[b])
        pltpu.async_copy(row_hbm.at[pl.ds(base, GROUP)], r_idx[b], sem_e[b])
        pltpu.async_copy(val_hbm.at[pl.ds(base, GROUP)], vv[b], sem_e[b])

    def wait_edge(b):
        pltpu.make_async_copy(col_hbm.at[pl.ds(0, GROUP)], c_idx[b], sem_e[b]).wait()
        pltpu.make_async_copy(col_hbm.at[pl.ds(0, GROUP)], r_idx[b], sem_e[b]).wait()
        pltpu.make_async_copy(val_hbm.at[pl.ds(0, GROUP)], vv[b], sem_e[b]).wait()

    def wait_gath(b):
        pltpu.make_async_copy(tab_hbm.at[c_idx[b]], rows[b], sem_g[b]).wait()

    def wait_scat(b):
        pltpu.make_async_copy(rows[b], acc_sh.at[sx[b]], sem_s[b]).wait()

    fire_edge(0, 0)
    wait_edge(0)
    pltpu.async_copy(tab_hbm.at[c_idx[0]], rows[0], sem_g[0])
    fire_edge(1, 1)

    def grp2(i, carry):
        for par in range(2):
            g = i * 2 + par
            b = par
            o = 1 - par

            @pl.when(g + 1 < cnt)
            def _():
                wait_edge(o)

                @pl.when(g >= 1)
                def _():
                    wait_scat(o)
                pltpu.async_copy(tab_hbm.at[c_idx[o]], rows[o], sem_g[o])

            @pl.when(g < cnt)
            def _():
                wait_gath(b)
                for k in range(GROUP // 16):
                    sl = pl.ds(k * 16, 16)
                    sx[b][sl] = r_idx[b][sl]
                for k in range(GROUP // 16):
                    v16 = vv[b][pl.ds(k * 16, 16)]
                    for m in range(16):
                        e = k * 16 + m
                        ve = v16[m]
                        rows[b][e, pl.ds(0, 16)] = rows[b][e, pl.ds(0, 16)] * ve
                        rows[b][e, pl.ds(16, 16)] = rows[b][e, pl.ds(16, 16)] * ve
                pltpu.async_copy(rows[b], acc_sh.at[sx[b]], sem_s[b], add=True)

            @pl.when(g + 2 < cnt)
            def _():
                fire_edge(g + 2, b)
        return carry
    lax.fori_loop(0, (G_BASE + 2) // 2, grp2, 0)

    wait_scat(0)
    wait_scat(1)
    plsc.subcore_barrier()
    pltpu.sync_copy(acc_sh.at[pl.ds(row0, rpt)], acc_out.at[c, pl.ds(row0, rpt)])


def _make_ws(rpt):
    nrows = rpt * NTILE
    def run(*args):
        pair = lambda sh, dt: (pltpu.VMEM(sh, dt), pltpu.VMEM(sh, dt))
        sems = lambda: (pltpu.SemaphoreType.DMA, pltpu.SemaphoreType.DMA)
        return pl.kernel(
            functools.partial(_ws_body, rpt),
            out_type=jax.ShapeDtypeStruct((2, nrows, HD), jnp.float32),
            mesh=_mesh(),
            compiler_params=pltpu.CompilerParams(use_tc_tiling_on_sc=False),
            scratch_types=[
                pair((GROUP,), jnp.int32),
                pair((GROUP,), jnp.int32),
                pair((GROUP,), jnp.float32),
                pair((GROUP, HD), jnp.float32),
                pair((GROUP,), jnp.int32),
                pltpu.VMEM_SHARED((nrows, HD), jnp.float32),
                sems(),
                sems(),
                sems(),
            ],
        )(*args)
    return run


_ws_ent = _make_ws(ENT_RPT)
_ws_tot = _make_ws(TOT_RPT)


def _gather_body(ifin_hbm, ufin_hbm, iid_hbm, uid_hbm, irows_out, urows_out,
                 idbuf, rowbuf, sem0):
    c = lax.axis_index("c")
    s = lax.axis_index("s")
    w = s * 2 + c
    base = w * (BATCH // 32)
    n = BATCH // 32
    pltpu.sync_copy(iid_hbm.at[pl.ds(base, n)], idbuf)
    pltpu.async_copy(ifin_hbm.at[idbuf], rowbuf, sem0).wait()
    pltpu.sync_copy(rowbuf, irows_out.at[pl.ds(base, n)])
    pltpu.sync_copy(uid_hbm.at[pl.ds(base, n)], idbuf)
    for k in range(n // 16):
        sl = pl.ds(k * 16, 16)
        idbuf[sl] = idbuf[sl] - N_ENT
    pltpu.async_copy(ufin_hbm.at[idbuf], rowbuf, sem0).wait()
    pltpu.sync_copy(rowbuf, urows_out.at[pl.ds(base, n)])


def _gatherk(*args):
    return pl.kernel(
        _gather_body,
        out_type=[
            jax.ShapeDtypeStruct((BATCH, D), jnp.float32),
            jax.ShapeDtypeStruct((BATCH, D), jnp.float32),
        ],
        mesh=_mesh(),
        compiler_params=pltpu.CompilerParams(use_tc_tiling_on_sc=False),
        scratch_types=[
            pltpu.VMEM((BATCH // 32,), jnp.int32),
            pltpu.VMEM((BATCH // 32, D), jnp.float32),
            pltpu.SemaphoreType.DMA,
        ],
    )(*args)


_RB = 2000


def _ct(a, b):
    return lax.dot_general(a, b, (((1,), (1,)), ((), ())),
                           preferred_element_type=jnp.float32)


def _prep_body(e_ref, rel_ref, wkw_ref, wkb_ref, t32_ref, h32_ref):
    q = jnp.dot(rel_ref[...], wkw_ref[...], preferred_element_type=jnp.float32)
    bias = _ct(wkb_ref[...], rel_ref[...])
    e = e_ref[...]
    t32_ref[...] = _ct(e, q[:, :D])
    h32_ref[...] = _ct(e, q[:, D:]) + bias


def _tc_prep(e_ent, rel, wkw, wkb_row):
    grid = N_ENT // _RB
    return pl.pallas_call(
        _prep_body,
        grid=(grid,),
        in_specs=[
            pl.BlockSpec((_RB, D), lambda i: (i, 0)),
            pl.BlockSpec((NREL, D), lambda i: (0, 0)),
            pl.BlockSpec((D, 2 * D), lambda i: (0, 0)),
            pl.BlockSpec((1, D), lambda i: (0, 0)),
        ],
        out_specs=[
            pl.BlockSpec((_RB, NREL), lambda i: (i, 0)),
            pl.BlockSpec((_RB, NREL), lambda i: (i, 0)),
        ],
        out_shape=[
            jax.ShapeDtypeStruct((N_ENT, NREL), jnp.float32),
            jax.ShapeDtypeStruct((N_ENT, NREL), jnp.float32),
        ],
    )(e_ent, rel, wkw, wkb_row)


def _ent_body(last, numlo_ref, numhi_ref, den0_ref, den1_ref, iglo_ref,
              ighi_ref, isum_ref, rel_ref, wkw_ref, wkb_ref, wa_ref, wb_ref,
              *out_refs):
    numlo = numlo_ref[0] + numlo_ref[1]
    numhi = numhi_ref[0] + numhi_ref[1]
    den = den0_ref[:, 0] + den1_ref[:, 0]
    inv = (1.0 / (den + 1e-16))[:, None]
    ekg_lo = numlo * inv
    ekg_hi = numhi * inv
    collab_lo = iglo_ref[0] + iglo_ref[1]
    collab_hi = ighi_ref[0] + ighi_ref[1]
    collab = jnp.concatenate([collab_lo, collab_hi], axis=1)
    if last:
        (isum_out,) = out_refs
        isum_out[...] = isum_ref[...] + collab
        return
    (elo_ref, ehi_ref, t32_ref, h32_ref, dlo_ref, dhi_ref, isum_out) = out_refs
    ekg = jnp.concatenate([ekg_lo, ekg_hi], axis=1)
    g = jax.nn.sigmoid(_ct(ekg, wa_ref[...]) + _ct(collab, wb_ref[...]))
    dual = g * ekg + (1.0 - g) * collab
    q = jnp.dot(rel_ref[...], wkw_ref[...], preferred_element_type=jnp.float32)
    bias = _ct(wkb_ref[...], rel_ref[...])
    elo_ref[...] = ekg_lo
    ehi_ref[...] = ekg_hi
    t32_ref[...] = _ct(ekg, q[:, :D])
    h32_ref[...] = _ct(ekg, q[:, D:]) + bias
    dlo_ref[...] = dual[:, :HD]
    dhi_ref[...] = dual[:, HD:]
    isum_out[...] = isum_ref[...] + collab


def _tc_entity(last, num_p, numhi_p, den_p0, den_p1, iglo_p, ighi_p, item_sum,
               rel, wkw, wkb_row, wa, wb):
    grid = N_ENT // _RB
    p3 = pl.BlockSpec((2, _RB, HD), lambda i: (0, i, 0))
    pden = pl.BlockSpec((_RB, 1), lambda i: (i, 0))
    full = lambda shape: pl.BlockSpec(shape, lambda i: tuple(0 for _ in shape))
    ob32 = pl.BlockSpec((_RB, HD), lambda i: (i, 0))
    ob64 = pl.BlockSpec((_RB, D), lambda i: (i, 0))
    if last:
        out_specs = [ob64]
        out_shape = [jax.ShapeDtypeStruct((N_ENT, D), jnp.float32)]
    else:
        out_specs = [ob32, ob32, ob32, ob32, ob32, ob32, ob64]
        out_shape = [
            jax.ShapeDtypeStruct((N_ENT, HD), jnp.float32),
            jax.ShapeDtypeStruct((N_ENT, HD), jnp.float32),
            jax.ShapeDtypeStruct((N_ENT, NREL), jnp.float32),
            jax.ShapeDtypeStruct((N_ENT, NREL), jnp.float32),
            jax.ShapeDtypeStruct((N_ENT, HD), jnp.float32),
            jax.ShapeDtypeStruct((N_ENT, HD), jnp.float32),
            jax.ShapeDtypeStruct((N_ENT, D), jnp.float32),
        ]
    return pl.pallas_call(
        functools.partial(_ent_body, last),
        grid=(grid,),
        in_specs=[
            p3, p3, pden, pden, p3, p3,
            ob64,
            full((NREL, D)), full((D, 2 * D)), full((1, D)),
            full((D, D)), full((D, D)),
        ],
        out_specs=out_specs,
        out_shape=out_shape,
    )(num_p, numhi_p, den_p0, den_p1, iglo_p, ighi_p, item_sum,
      rel, wkw, wkb_row, wa, wb)


def _usr_body(iglo_ref, ighi_ref, usum_ref, ulo_ref, uhi_ref, usum_out):
    lo = iglo_ref[0] + iglo_ref[1]
    hi = ighi_ref[0] + ighi_ref[1]
    ulo_ref[...] = lo
    uhi_ref[...] = hi
    usum_out[...] = usum_ref[...] + jnp.concatenate([lo, hi], axis=1)


def _tc_user(iglo_p, ighi_p, user_sum):
    grid = N_USR // _RB
    ent_blocks = N_ENT // _RB
    p3 = pl.BlockSpec((2, _RB, HD), lambda i: (0, ent_blocks + i, 0))
    ob32 = pl.BlockSpec((_RB, HD), lambda i: (i, 0))
    ob64 = pl.BlockSpec((_RB, D), lambda i: (i, 0))
    return pl.pallas_call(
        _usr_body,
        grid=(grid,),
        in_specs=[p3, p3, ob64],
        out_specs=[ob32, ob32, ob64],
        out_shape=[
            jax.ShapeDtypeStruct((N_USR, HD), jnp.float32),
            jax.ShapeDtypeStruct((N_USR, HD), jnp.float32),
            jax.ShapeDtypeStruct((N_USR, D), jnp.float32),
        ],
    )(iglo_p, ighi_p, user_sum)


def _scores_body(u_ref, i_ref, o_ref):
    o_ref[...] = _ct(u_ref[...], i_ref[...])


def _tc_scores(user_rows, item_rows):
    return pl.pallas_call(
        _scores_body,
        out_shape=jax.ShapeDtypeStruct((BATCH, BATCH), jnp.float32),
    )(user_rows, item_rows)


def kernel(h_list, t_list, r_list, ai_row, ai_col, ai_val, user_ids, item_ids,
           entity_user_embed, relation_embed, Wk_w, Wk_b, Wa, Wb):
    e_ent = entity_user_embed[:N_ENT]
    wkb_row = Wk_b.reshape(1, D)
    T32, H32 = _tc_prep(e_ent, relation_embed, Wk_w, wkb_row)
    e_lo = e_ent[:, :HD]
    e_hi = e_ent[:, HD:]
    ig_lo = entity_user_embed[:, :HD]
    ig_hi = entity_user_embed[:, HD:]
    item_sum = e_ent
    user_sum = entity_user_embed[N_ENT:]
    for layer in range(2):
        last = layer == 1
        w_e, numlo_p, den_p0, den_p1 = _kg1(h_list, t_list, r_list,
                                            T32.reshape(-1), H32.reshape(-1),
                                            e_lo)
        numhi_p = _ws_ent(t_list, h_list, w_e, e_hi)
        iglo_p = _ws_tot(ai_col, ai_row, ai_val, ig_lo)
        ighi_p = _ws_tot(ai_col, ai_row, ai_val, ig_hi)
        ent_out = _tc_entity(last, numlo_p, numhi_p,
                             den_p0.reshape(N_DEN, 1), den_p1.reshape(N_DEN, 1),
                             iglo_p, ighi_p, item_sum,
                             relation_embed, Wk_w, wkb_row, Wa, Wb)
        u_lo, u_hi, user_sum = _tc_user(iglo_p, ighi_p, user_sum)
        if last:
            (item_sum,) = ent_out
        else:
            e_lo, e_hi, T32, H32, d_lo, d_hi, item_sum = ent_out
            ig_lo = jnp.concatenate([d_lo, u_lo], axis=0)
            ig_hi = jnp.concatenate([d_hi, u_hi], axis=0)
    item_rows, user_rows = _gatherk(item_sum, user_sum, item_ids, user_ids)
    return _tc_scores(user_rows, item_rows)
